# Initial kernel scaffold; baseline (speedup 1.0000x reference)
import jax
import jax.numpy as jnp
from jax.experimental import pallas as pl


def kernel(x, edge_index, W, b):
    raise NotImplementedError("write your pallas kernel here")


# speedup vs baseline: 1233.8594x; 1233.8594x over previous
import functools

import jax
import jax.numpy as jnp
from jax import lax
from jax.experimental import pallas as pl
from jax.experimental.pallas import tpu as pltpu
from jax.experimental.pallas import tpu_sc as plsc

N = 10000
C = 128
K = 32
NW = 32
NPW = 320
NPAD = NW * NPW
CH = 4
NCHUNK = NPW // CH
BLK = 1024
NLANE = 16
NVEC = C // NLANE


def _tables_body(x_ref, w_ref, b_ref, a_ref, b2_ref):
    x = x_ref[...]
    w = w_ref[...]
    w1 = w[:, :C]
    w2 = w[:, C:]
    dn = (((0,), (1,)), ((), ()))
    a = lax.dot_general(x, w1 - w2, dn, preferred_element_type=jnp.float32)
    a_ref[...] = a + b_ref[...]
    b2_ref[...] = lax.dot_general(x, w2, dn, preferred_element_type=jnp.float32)


_tc_tables = pl.pallas_call(
    _tables_body,
    grid=(NPAD // BLK,),
    in_specs=[
        pl.BlockSpec((C, BLK), lambda i: (0, i)),
        pl.BlockSpec((C, 2 * C), lambda i: (0, 0)),
        pl.BlockSpec((1, C), lambda i: (0, 0)),
    ],
    out_specs=[
        pl.BlockSpec((BLK, C), lambda i: (i, 0)),
        pl.BlockSpec((BLK, C), lambda i: (i, 0)),
    ],
    out_shape=[jax.ShapeDtypeStruct((NPAD, C), jnp.float32)] * 2,
)


def _tr_body(i_ref, o_ref):
    o_ref[...] = i_ref[...].T


_tc_transpose = pl.pallas_call(
    _tr_body,
    grid=(NPAD // BLK,),
    in_specs=[pl.BlockSpec((BLK, C), lambda i: (i, 0))],
    out_specs=pl.BlockSpec((C, BLK), lambda i: (0, i)),
    out_shape=jax.ShapeDtypeStruct((C, NPAD), jnp.float32),
)


def _sc_body(at_hbm, bt_hbm, idx1_hbm, idx0_hbm, out_hbm,
             idx1_v, idx0_v, arows, brows, ostage, sem_a, sem_b):
    wid = lax.axis_index("s") * 2 + lax.axis_index("c")
    base = wid * NPW
    pltpu.sync_copy(idx1_hbm.at[pl.ds(base * K, NPW * K)], idx1_v)
    pltpu.sync_copy(idx0_hbm.at[pl.ds(base * K, NPW * K)], idx0_v)

    def chunk_body(ci, carry):
        off = ci * (CH * K)
        cp_a = pltpu.async_copy(at_hbm.at[idx1_v.at[pl.ds(off, CH * K)]],
                                arows, sem_a)
        cp_b = pltpu.async_copy(bt_hbm.at[idx0_v.at[pl.ds(off, CH * K)]],
                                brows, sem_b)
        cp_a.wait()
        cp_b.wait()
        for v in range(CH):
            def kstep(k, accs, _v=v):
                row = _v * K + k
                return tuple(
                    jnp.maximum(accs[l],
                                arows[row, pl.ds(l * NLANE, NLANE)]
                                + brows[row, pl.ds(l * NLANE, NLANE)])
                    for l in range(NVEC))
            init = tuple(jnp.full((NLANE,), -jnp.inf, jnp.float32)
                         for _ in range(NVEC))
            accs = lax.fori_loop(0, K, kstep, init)
            for l in range(NVEC):
                ostage[v, pl.ds(l * NLANE, NLANE)] = jnp.maximum(accs[l], 0.0)
        pltpu.sync_copy(ostage, out_hbm.at[pl.ds(base + ci * CH, CH)])
        return carry

    lax.fori_loop(0, NCHUNK, chunk_body, 0)


_sc_gather_max = functools.partial(
    pl.kernel,
    out_type=jax.ShapeDtypeStruct((NPAD, C), jnp.float32),
    mesh=plsc.VectorSubcoreMesh(core_axis_name="c", subcore_axis_name="s"),
    scratch_types=[
        pltpu.VMEM((NPW * K,), jnp.int32),
        pltpu.VMEM((NPW * K,), jnp.int32),
        pltpu.VMEM((CH * K, C), jnp.float32),
        pltpu.VMEM((CH * K, C), jnp.float32),
        pltpu.VMEM((CH, C), jnp.float32),
        pltpu.SemaphoreType.DMA,
        pltpu.SemaphoreType.DMA,
    ],
)(_sc_body)


def kernel(x, edge_index, W, b):
    X = x[0, :, :, 0]
    Xp = jnp.pad(X, ((0, 0), (0, NPAD - N)))
    idx1 = jnp.pad(edge_index[1, 0].reshape(-1), (0, (NPAD - N) * K))
    idx0 = jnp.pad(edge_index[0, 0].reshape(-1), (0, (NPAD - N) * K))
    at, bt = _tc_tables(Xp, W, b.reshape(1, C))
    out_t = _sc_gather_max(at, bt, idx1, idx0)
    yt = _tc_transpose(out_t)
    return yt[:, :N][None, :, :, None]
